# baseline (speedup 1.0000x reference)
import dataclasses
import functools

import jax
import jax.numpy as jnp
from jax import lax
from jax.experimental import pallas as pl
from jax.experimental.pallas import tpu as pltpu
from jax.experimental.pallas import tpu_sc as plsc

BATCH = 4096
N_IN = 2048
H1 = 1024
H2 = 512
K_SPARSE = int(H2 * 0.05)

BLOCK_B = 512

NC = 2
NS = 16
NW = NC * NS
ROWS_PER_W = BATCH // NW
N_CHUNK = H2 // 16


def _encoder_kernel(x_ref, we1_ref, be1_ref, we2_ref, be2_ref,
                    h2_ref, h2t_ref):
    x = x_ref[...].astype(jnp.bfloat16)
    h1 = lax.dot_general(
        x, we1_ref[...].astype(jnp.bfloat16), (((1,), (1,)), ((), ())),
        preferred_element_type=jnp.float32)
    h1 = jax.nn.sigmoid(h1 + be1_ref[...]).astype(jnp.bfloat16)
    h2 = lax.dot_general(
        h1, we2_ref[...].astype(jnp.bfloat16), (((1,), (1,)), ((), ())),
        preferred_element_type=jnp.float32)
    h2 = jax.nn.sigmoid(h2 + be2_ref[...])
    h2_ref[...] = h2
    h2t_ref[...] = h2.T


def _decoder_kernel(h2_ref, thr_ref, wd1_ref, bd1_ref, wd0_ref, bd0_ref,
                    out_ref):
    h2 = h2_ref[...]
    h2m = jnp.where(h2 >= thr_ref[...], h2, 0.0)
    d1 = lax.dot_general(
        h2m.astype(jnp.bfloat16), wd1_ref[...].astype(jnp.bfloat16),
        (((1,), (1,)), ((), ())),
        preferred_element_type=jnp.float32)
    d1 = jax.nn.sigmoid(d1 + bd1_ref[...]).astype(jnp.bfloat16)
    out = lax.dot_general(
        d1, wd0_ref[...].astype(jnp.bfloat16), (((1,), (1,)), ((), ())),
        preferred_element_type=jnp.float32)
    out_ref[...] = out + bd0_ref[...]


def _sc_threshold_body(h2t_hbm, thr_hbm, slab, thr_v, sem):
    c = lax.axis_index("c")
    s = lax.axis_index("s")
    wid = s * NC + c
    base = wid * ROWS_PER_W
    pltpu.async_copy(h2t_hbm.at[:, pl.ds(base, ROWS_PER_W)], slab, sem).wait()

    lanes = lax.iota(jnp.int32, 16)
    neg_inf = jnp.float32(-jnp.inf)

    @pl.loop(0, ROWS_PER_W // 16)
    def _group(g):
        col = g * 16
        cols = col + lanes
        m = []
        for ch in range(N_CHUNK):
            acc = slab[16 * ch, pl.ds(col, 16)]
            for p in range(1, 16):
                acc = jnp.maximum(acc, slab[16 * ch + p, pl.ds(col, 16)])
            m.append(acc)

        gmax = m[0]
        for it in range(K_SPARSE):
            t1 = [jnp.maximum(m[2 * i], m[2 * i + 1]) for i in range(16)]
            t2 = [jnp.maximum(t1[2 * i], t1[2 * i + 1]) for i in range(8)]
            t3 = [jnp.maximum(t2[2 * i], t2[2 * i + 1]) for i in range(4)]
            t4 = [jnp.maximum(t3[0], t3[1]), jnp.maximum(t3[2], t3[3])]
            gmax = jnp.maximum(t4[0], t4[1])
            if it == K_SPARSE - 1:
                break
            cidx = jnp.full((16,), N_CHUNK, jnp.int32)
            for ch in range(N_CHUNK - 1, -1, -1):
                cidx = jnp.where(m[ch] == gmax, ch, cidx)
            rows = cidx * 16
            vg = [plsc.load_gather(slab, [rows + p, cols])
                  for p in range(16)]
            mg = [jnp.where(v < gmax, v, neg_inf) for v in vg]
            for step in (8, 4, 2, 1):
                mg = [jnp.maximum(mg[i], mg[i + step]) for i in range(step)]
            newmax = mg[0]
            m = [jnp.where(cidx == ch, newmax, m[ch])
                 for ch in range(N_CHUNK)]
        thr_v[pl.ds(col, 16)] = gmax

    pltpu.sync_copy(thr_v, thr_hbm.at[pl.ds(base, ROWS_PER_W)])


def _sc_threshold(h2t):
    mesh = plsc.VectorSubcoreMesh(core_axis_name="c", subcore_axis_name="s")
    cp = pltpu.CompilerParams()
    if "needs_layout_passes" in pltpu.CompilerParams.__dataclass_fields__:
        cp = dataclasses.replace(cp, needs_layout_passes=False)
    return pl.kernel(
        _sc_threshold_body,
        mesh=mesh,
        out_type=jax.ShapeDtypeStruct((BATCH,), jnp.float32),
        scratch_types=[
            pltpu.VMEM((H2, ROWS_PER_W), jnp.float32),
            pltpu.VMEM((ROWS_PER_W,), jnp.float32),
            pltpu.SemaphoreType.DMA,
        ],
        compiler_params=cp,
    )(h2t)


@jax.jit
def kernel(input, We1, be1, We2, be2, Wd0, bd0, Wd1, bd1):
    b1 = be1.reshape(1, H1)
    b2 = be2.reshape(1, H2)
    b0 = bd0.reshape(1, N_IN)
    bd1r = bd1.reshape(1, H1)
    const = lambda i: (0, 0)

    h2, h2t = pl.pallas_call(
        _encoder_kernel,
        grid=(BATCH // BLOCK_B,),
        in_specs=[
            pl.BlockSpec((BLOCK_B, N_IN), lambda i: (i, 0)),
            pl.BlockSpec((H1, N_IN), const),
            pl.BlockSpec((1, H1), const),
            pl.BlockSpec((H2, H1), const),
            pl.BlockSpec((1, H2), const),
        ],
        out_specs=[
            pl.BlockSpec((BLOCK_B, H2), lambda i: (i, 0)),
            pl.BlockSpec((H2, BLOCK_B), lambda i: (0, i)),
        ],
        out_shape=[
            jax.ShapeDtypeStruct((BATCH, H2), jnp.float32),
            jax.ShapeDtypeStruct((H2, BATCH), jnp.float32),
        ],
    )(input, We1, b1, We2, b2)

    thr = _sc_threshold(h2t).reshape(BATCH, 1)

    out = pl.pallas_call(
        _decoder_kernel,
        grid=(BATCH // BLOCK_B,),
        in_specs=[
            pl.BlockSpec((BLOCK_B, H2), lambda i: (i, 0)),
            pl.BlockSpec((BLOCK_B, 1), lambda i: (i, 0)),
            pl.BlockSpec((H1, H2), const),
            pl.BlockSpec((1, H1), const),
            pl.BlockSpec((N_IN, H1), const),
            pl.BlockSpec((1, N_IN), const),
        ],
        out_specs=pl.BlockSpec((BLOCK_B, N_IN), lambda i: (i, 0)),
        out_shape=jax.ShapeDtypeStruct((BATCH, N_IN), jnp.float32),
    )(h2, thr, Wd1, bd1r, Wd0, b0)
    return out


# speedup vs baseline: 1.1111x; 1.1111x over previous
import dataclasses
import functools

import jax
import jax.numpy as jnp
from jax import lax
from jax.experimental import pallas as pl
from jax.experimental.pallas import tpu as pltpu
from jax.experimental.pallas import tpu_sc as plsc

BATCH = 4096
N_IN = 2048
H1 = 1024
H2 = 512
K_SPARSE = int(H2 * 0.05)

BLOCK_B = 512

NC = 2
NS = 16
NW = NC * NS
ROWS_PER_W = BATCH // NW
N_CHUNK = H2 // 16


def _encoder_kernel(x_ref, we1_ref, be1_ref, we2_ref, be2_ref, h2t_ref):
    x = x_ref[...].astype(jnp.bfloat16)
    h1 = lax.dot_general(
        x, we1_ref[...].astype(jnp.bfloat16), (((1,), (1,)), ((), ())),
        preferred_element_type=jnp.float32)
    h1 = jax.nn.sigmoid(h1 + be1_ref[...]).astype(jnp.bfloat16)
    h2 = lax.dot_general(
        h1, we2_ref[...].astype(jnp.bfloat16), (((1,), (1,)), ((), ())),
        preferred_element_type=jnp.float32)
    h2 = jax.nn.sigmoid(h2 + be2_ref[...])
    h2t_ref[...] = h2.T


def _decoder_kernel(h2t_ref, thr_ref, wd1_ref, bd1_ref, wd0_ref, bd0_ref,
                    out_ref):
    h2 = h2t_ref[...].T
    h2m = jnp.where(h2 >= thr_ref[...], h2, 0.0)
    d1 = lax.dot_general(
        h2m.astype(jnp.bfloat16), wd1_ref[...].astype(jnp.bfloat16),
        (((1,), (1,)), ((), ())),
        preferred_element_type=jnp.float32)
    d1 = jax.nn.sigmoid(d1 + bd1_ref[...]).astype(jnp.bfloat16)
    out = lax.dot_general(
        d1, wd0_ref[...].astype(jnp.bfloat16), (((1,), (1,)), ((), ())),
        preferred_element_type=jnp.float32)
    out_ref[...] = out + bd0_ref[...]


def _sc_threshold_body(h2t_hbm, thr_hbm, slab, thr_v, sem):
    c = lax.axis_index("c")
    s = lax.axis_index("s")
    wid = s * NC + c
    base = wid * ROWS_PER_W
    pltpu.async_copy(h2t_hbm.at[:, pl.ds(base, ROWS_PER_W)], slab, sem).wait()

    lanes = lax.iota(jnp.int32, 16)
    neg_inf = jnp.float32(-jnp.inf)

    @pl.loop(0, ROWS_PER_W // 16)
    def _group(g):
        col = g * 16
        cols = col + lanes
        m = []
        for ch in range(N_CHUNK):
            acc = slab[16 * ch, pl.ds(col, 16)]
            for p in range(1, 16):
                acc = jnp.maximum(acc, slab[16 * ch + p, pl.ds(col, 16)])
            m.append(acc)

        def step(_, carry):
            t, m = carry[0], list(carry[1:])
            gmax = m[0]
            for v in m[1:]:
                gmax = jnp.maximum(gmax, v)
            cidx = jnp.full((16,), N_CHUNK, jnp.int32)
            for ch in range(N_CHUNK - 1, -1, -1):
                cidx = jnp.where(m[ch] == gmax, ch, cidx)
            rows = cidx * 16
            newmax = jnp.full((16,), neg_inf, jnp.float32)
            for p in range(16):
                v = plsc.load_gather(slab, [rows + p, cols])
                newmax = jnp.maximum(newmax, jnp.where(v < gmax, v, neg_inf))
            m = [jnp.where(cidx == ch, newmax, m[ch])
                 for ch in range(N_CHUNK)]
            return (gmax, *m)

        init = (jnp.full((16,), jnp.inf, jnp.float32), *m)
        final = lax.fori_loop(0, K_SPARSE, step, init)
        thr_v[pl.ds(col, 16)] = final[0]

    pltpu.sync_copy(thr_v, thr_hbm.at[pl.ds(base, ROWS_PER_W)])


def _sc_threshold(h2t):
    mesh = plsc.VectorSubcoreMesh(core_axis_name="c", subcore_axis_name="s")
    cp = pltpu.CompilerParams()
    if "needs_layout_passes" in pltpu.CompilerParams.__dataclass_fields__:
        cp = dataclasses.replace(cp, needs_layout_passes=False)
    return pl.kernel(
        _sc_threshold_body,
        mesh=mesh,
        out_type=jax.ShapeDtypeStruct((BATCH,), jnp.float32),
        scratch_types=[
            pltpu.VMEM((H2, ROWS_PER_W), jnp.float32),
            pltpu.VMEM((ROWS_PER_W,), jnp.float32),
            pltpu.SemaphoreType.DMA,
        ],
        compiler_params=cp,
    )(h2t)


@jax.jit
def kernel(input, We1, be1, We2, be2, Wd0, bd0, Wd1, bd1):
    b1 = be1.reshape(1, H1)
    b2 = be2.reshape(1, H2)
    b0 = bd0.reshape(1, N_IN)
    bd1r = bd1.reshape(1, H1)
    const = lambda i: (0, 0)

    h2t = pl.pallas_call(
        _encoder_kernel,
        grid=(BATCH // BLOCK_B,),
        in_specs=[
            pl.BlockSpec((BLOCK_B, N_IN), lambda i: (i, 0)),
            pl.BlockSpec((H1, N_IN), const),
            pl.BlockSpec((1, H1), const),
            pl.BlockSpec((H2, H1), const),
            pl.BlockSpec((1, H2), const),
        ],
        out_specs=pl.BlockSpec((H2, BLOCK_B), lambda i: (0, i)),
        out_shape=jax.ShapeDtypeStruct((H2, BATCH), jnp.float32),
    )(input, We1, b1, We2, b2)

    thr = _sc_threshold(h2t).reshape(BATCH, 1)

    out = pl.pallas_call(
        _decoder_kernel,
        grid=(BATCH // BLOCK_B,),
        in_specs=[
            pl.BlockSpec((H2, BLOCK_B), lambda i: (0, i)),
            pl.BlockSpec((BLOCK_B, 1), lambda i: (i, 0)),
            pl.BlockSpec((H1, H2), const),
            pl.BlockSpec((1, H1), const),
            pl.BlockSpec((N_IN, H1), const),
            pl.BlockSpec((1, N_IN), const),
        ],
        out_specs=pl.BlockSpec((BLOCK_B, N_IN), lambda i: (i, 0)),
        out_shape=jax.ShapeDtypeStruct((BATCH, N_IN), jnp.float32),
    )(h2t, thr, Wd1, bd1r, Wd0, b0)
    return out


# speedup vs baseline: 1.1163x; 1.0046x over previous
import dataclasses
import functools

import jax
import jax.numpy as jnp
from jax import lax
from jax.experimental import pallas as pl
from jax.experimental.pallas import tpu as pltpu
from jax.experimental.pallas import tpu_sc as plsc

BATCH = 4096
N_IN = 2048
H1 = 1024
H2 = 512
K_SPARSE = int(H2 * 0.05)

BLOCK_B = 512

NC = 2
NS = 16
NW = NC * NS
ROWS_PER_W = BATCH // NW
N_CHUNK = H2 // 16


def _encoder_kernel(x_ref, we1_ref, be1_ref, we2_ref, be2_ref,
                    h2_ref, h2t_ref):
    x = x_ref[...].astype(jnp.bfloat16)
    h1 = lax.dot_general(
        x, we1_ref[...].astype(jnp.bfloat16), (((1,), (1,)), ((), ())),
        preferred_element_type=jnp.float32)
    h1 = jax.nn.sigmoid(h1 + be1_ref[...]).astype(jnp.bfloat16)
    h2 = lax.dot_general(
        h1, we2_ref[...].astype(jnp.bfloat16), (((1,), (1,)), ((), ())),
        preferred_element_type=jnp.float32)
    h2 = jax.nn.sigmoid(h2 + be2_ref[...])
    h2_ref[...] = h2
    h2t_ref[...] = h2.T


def _decoder_kernel(h2_ref, thr_ref, wd1_ref, bd1_ref, wd0_ref, bd0_ref,
                    out_ref):
    h2 = h2_ref[...]
    h2m = jnp.where(h2 >= thr_ref[...], h2, 0.0)
    d1 = lax.dot_general(
        h2m.astype(jnp.bfloat16), wd1_ref[...].astype(jnp.bfloat16),
        (((1,), (1,)), ((), ())),
        preferred_element_type=jnp.float32)
    d1 = jax.nn.sigmoid(d1 + bd1_ref[...]).astype(jnp.bfloat16)
    out = lax.dot_general(
        d1, wd0_ref[...].astype(jnp.bfloat16), (((1,), (1,)), ((), ())),
        preferred_element_type=jnp.float32)
    out_ref[...] = out + bd0_ref[...]


def _sc_threshold_body(h2t_hbm, thr_hbm, slab, thr_v, sem):
    c = lax.axis_index("c")
    s = lax.axis_index("s")
    wid = s * NC + c
    base = wid * ROWS_PER_W
    pltpu.async_copy(h2t_hbm.at[:, pl.ds(base, ROWS_PER_W)], slab, sem).wait()

    lanes = lax.iota(jnp.int32, 16)
    neg_inf = jnp.float32(-jnp.inf)

    @pl.loop(0, ROWS_PER_W // 16)
    def _group(g):
        col = g * 16
        cols = col + lanes
        m = []
        for ch in range(N_CHUNK):
            acc = slab[16 * ch, pl.ds(col, 16)]
            for p in range(1, 16):
                acc = jnp.maximum(acc, slab[16 * ch + p, pl.ds(col, 16)])
            m.append(acc)

        def step(_, carry):
            t, m = carry[0], list(carry[1:])
            gmax = m[0]
            for v in m[1:]:
                gmax = jnp.maximum(gmax, v)
            cidx = jnp.full((16,), N_CHUNK, jnp.int32)
            for ch in range(N_CHUNK - 1, -1, -1):
                cidx = jnp.where(m[ch] == gmax, ch, cidx)
            rows = cidx * 16
            newmax = jnp.full((16,), neg_inf, jnp.float32)
            for p in range(16):
                v = plsc.load_gather(slab, [rows + p, cols])
                newmax = jnp.maximum(newmax, jnp.where(v < gmax, v, neg_inf))
            m = [jnp.where(cidx == ch, newmax, m[ch])
                 for ch in range(N_CHUNK)]
            return (gmax, *m)

        init = (jnp.full((16,), jnp.inf, jnp.float32), *m)
        final = lax.fori_loop(0, K_SPARSE, step, init)
        thr_v[pl.ds(col, 16)] = final[0]

    pltpu.sync_copy(thr_v, thr_hbm.at[pl.ds(base, ROWS_PER_W)])


def _sc_threshold(h2t):
    mesh = plsc.VectorSubcoreMesh(core_axis_name="c", subcore_axis_name="s")
    cp = pltpu.CompilerParams()
    if "needs_layout_passes" in pltpu.CompilerParams.__dataclass_fields__:
        cp = dataclasses.replace(cp, needs_layout_passes=False)
    return pl.kernel(
        _sc_threshold_body,
        mesh=mesh,
        out_type=jax.ShapeDtypeStruct((BATCH,), jnp.float32),
        scratch_types=[
            pltpu.VMEM((H2, ROWS_PER_W), jnp.float32),
            pltpu.VMEM((ROWS_PER_W,), jnp.float32),
            pltpu.SemaphoreType.DMA,
        ],
        compiler_params=cp,
    )(h2t)


@jax.jit
def kernel(input, We1, be1, We2, be2, Wd0, bd0, Wd1, bd1):
    b1 = be1.reshape(1, H1)
    b2 = be2.reshape(1, H2)
    b0 = bd0.reshape(1, N_IN)
    bd1r = bd1.reshape(1, H1)
    const = lambda i: (0, 0)

    h2, h2t = pl.pallas_call(
        _encoder_kernel,
        grid=(BATCH // BLOCK_B,),
        in_specs=[
            pl.BlockSpec((BLOCK_B, N_IN), lambda i: (i, 0)),
            pl.BlockSpec((H1, N_IN), const),
            pl.BlockSpec((1, H1), const),
            pl.BlockSpec((H2, H1), const),
            pl.BlockSpec((1, H2), const),
        ],
        out_specs=[
            pl.BlockSpec((BLOCK_B, H2), lambda i: (i, 0)),
            pl.BlockSpec((H2, BLOCK_B), lambda i: (0, i)),
        ],
        out_shape=[
            jax.ShapeDtypeStruct((BATCH, H2), jnp.float32),
            jax.ShapeDtypeStruct((H2, BATCH), jnp.float32),
        ],
    )(input, We1, b1, We2, b2)

    thr = _sc_threshold(h2t).reshape(BATCH, 1)

    out = pl.pallas_call(
        _decoder_kernel,
        grid=(BATCH // BLOCK_B,),
        in_specs=[
            pl.BlockSpec((BLOCK_B, H2), lambda i: (i, 0)),
            pl.BlockSpec((BLOCK_B, 1), lambda i: (i, 0)),
            pl.BlockSpec((H1, H2), const),
            pl.BlockSpec((1, H1), const),
            pl.BlockSpec((N_IN, H1), const),
            pl.BlockSpec((1, N_IN), const),
        ],
        out_specs=pl.BlockSpec((BLOCK_B, N_IN), lambda i: (i, 0)),
        out_shape=jax.ShapeDtypeStruct((BATCH, N_IN), jnp.float32),
    )(h2, thr, Wd1, bd1r, Wd0, b0)
    return out
